# baseline (speedup 1.0000x reference)
import functools

import jax
import jax.numpy as jnp
from jax import lax
from jax.experimental import pallas as pl
from jax.experimental.pallas import tpu as pltpu
from jax.experimental.pallas import tpu_sc as plsc

NXYZ = 48
G = NXYZ * NXYZ * NXYZ
CELLS = 128
NC, NS, L = 2, 16, 16
NW = NC * NS
CPW = CELLS // NW
RAD = 4
NEG = -1e9
THRESH = -1e8
NSTEP = G // L
NCH = 16
CH_STEP = NSTEP // NCH
CH = CH_STEP * L
DPAD = 16
BIG = 1 << 30
UNROLL = 8


def _merge(cms, cis):
    cm, ci = cms[0], cis[0]
    for g in range(1, len(cms)):
        m = jnp.maximum(cm, cms[g])
        both = (cm == m) & (cms[g] == m)
        pick_b = (cms[g] == m) & ~(cm == m)
        ni = jnp.where(both, jnp.minimum(ci, cis[g]),
                       jnp.where(pick_b, cis[g], ci))
        cm, ci = m, ni
    return cm, ci


def _body(dens, maskf, grid, out, vbuf, mb0, mb1, idxv, xyzv, outbuf,
          semd, semm0, semm1, semg):
    cid = lax.axis_index("c")
    sid = lax.axis_index("s")
    w = sid * NC + cid
    _worker(w, dens, maskf, grid, out, vbuf, mb0, mb1, idxv, xyzv, outbuf,
            semd, semm0, semm1, semg)


def _worker(w, dens, maskf, grid, out, vbuf, mb0, mb1, idxv, xyzv, outbuf,
            semd, semm0, semm1, semg):
    lane = lax.iota(jnp.int32, L)
    neg = jnp.float32(NEG)
    thresh = jnp.float32(THRESH)
    big = jnp.int32(BIG)
    gidxs, scores, alives = [], [], []

    for t in range(CPW):
        cell = w * CPW + t
        cpd = pltpu.async_copy(dens.at[pl.ds(cell * G, G)], vbuf, semd)
        mcopies = [None, None]
        mcopies[0] = pltpu.async_copy(maskf.at[pl.ds(0, CH)], mb0, semm0)
        cpd.wait()

        curmax = tuple(jnp.full((L,), neg, jnp.float32) for _ in range(UNROLL))
        curidx = tuple(jnp.zeros((L,), jnp.int32) for _ in range(UNROLL))
        for ch in range(NCH):
            mcopies[ch % 2].wait()
            if ch + 1 < NCH:
                mb_next = mb1 if (ch + 1) % 2 else mb0
                sem_next = semm1 if (ch + 1) % 2 else semm0
                mcopies[(ch + 1) % 2] = pltpu.async_copy(
                    maskf.at[pl.ds((ch + 1) * CH, CH)], mb_next, sem_next)
            mb = mb1 if ch % 2 else mb0

            def step(u, carry, _ch=ch, _mb=mb):
                cms, cis = carry
                ncms, ncis = [], []
                for g in range(UNROLL):
                    off = _ch * CH + (u * UNROLL + g) * L
                    moff = (u * UNROLL + g) * L
                    v = vbuf[pl.ds(off, L)]
                    m = _mb[pl.ds(moff, L)]
                    v = jnp.where(m > jnp.float32(0.5), v, neg)
                    vbuf[pl.ds(off, L)] = v
                    gt = v > cms[g]
                    ncms.append(jnp.where(gt, v, cms[g]))
                    ncis.append(jnp.where(gt, lane + off, cis[g]))
                return tuple(ncms), tuple(ncis)

            curmax, curidx = lax.fori_loop(
                0, CH_STEP // UNROLL, step, (curmax, curidx))

        cm1, ci1 = _merge(curmax, curidx)
        gmax = jnp.max(cm1)
        gidx = jnp.min(jnp.where(cm1 == gmax, ci1, big))
        alive1 = gmax >= thresh

        i0 = gidx // (NXYZ * NXYZ)
        rem = gidx - i0 * (NXYZ * NXYZ)
        j0 = rem // NXYZ
        k0 = rem - j0 * NXYZ
        ks = jnp.clip(k0 - RAD, 0, NXYZ - L)
        kwin = ks + lane
        kmask = jnp.abs(kwin - k0) <= RAD

        def supp(s, carry):
            di = s // (2 * RAD + 1) - RAD
            dj = s % (2 * RAD + 1) - RAD
            ii = i0 + di
            jj = j0 + dj
            valid = (ii >= 0) & (ii < NXYZ) & (jj >= 0) & (jj < NXYZ)
            iic = jnp.clip(ii, 0, NXYZ - 1)
            jjc = jnp.clip(jj, 0, NXYZ - 1)
            base = (iic * NXYZ + jjc) * NXYZ + ks
            vv = vbuf[pl.ds(base, L)]
            vbuf[pl.ds(base, L)] = jnp.where(kmask & valid, neg, vv)
            return carry

        lax.fori_loop(0, (2 * RAD + 1) * (2 * RAD + 1), supp, 0)

        def step2(u, carry):
            cms, cis = carry
            ncms, ncis = [], []
            for g in range(UNROLL):
                off = (u * UNROLL + g) * L
                v = vbuf[pl.ds(off, L)]
                gt = v > cms[g]
                ncms.append(jnp.where(gt, v, cms[g]))
                ncis.append(jnp.where(gt, lane + off, cis[g]))
            return tuple(ncms), tuple(ncis)

        curmax2 = tuple(jnp.full((L,), neg, jnp.float32) for _ in range(UNROLL))
        curidx2 = tuple(jnp.zeros((L,), jnp.int32) for _ in range(UNROLL))
        curmax2, curidx2 = lax.fori_loop(0, NSTEP // UNROLL, step2,
                                         (curmax2, curidx2))
        cm2, ci2 = _merge(curmax2, curidx2)
        gmax2 = jnp.max(cm2)
        gidx2 = jnp.min(jnp.where(cm2 == gmax2, ci2, big))
        alive2 = alive1 & (gmax2 >= thresh)

        gidxs.append(jnp.where(alive1, gidx, 0))
        gidxs.append(jnp.where(alive2, gidx2, 0))
        scores.append(jnp.where(alive1, gmax, neg))
        scores.append(jnp.where(alive2, gmax2, neg))
        alives.append(alive1)
        alives.append(alive2)

    idxacc = jnp.zeros((L,), jnp.int32)
    for p in range(2 * CPW):
        idxacc = jnp.where(lane == p, gidxs[p], idxacc)
    idxv[...] = idxacc
    pltpu.async_copy(grid.at[idxv], xyzv, semg).wait()

    afl = [jnp.where(a, jnp.float32(1.0), jnp.float32(0.0)) for a in alives]
    v0 = jnp.zeros((L,), jnp.float32)
    v1 = jnp.zeros((L,), jnp.float32)
    v2 = jnp.zeros((L,), jnp.float32)
    for p in range(2 * CPW):
        row = xyzv[p, :]
        for q in range(3):
            s = 3 * p + q
            val = row[q] * afl[p]
            if s < 16:
                v0 = jnp.where(lane == s, val, v0)
            else:
                v1 = jnp.where(lane == (s - 16), val, v1)
    for p in range(2 * CPW):
        v1 = jnp.where(lane == (8 + p), scores[p], v1)
        v2 = jnp.where(lane == p, afl[p], v2)
    outbuf[pl.ds(0, L)] = v0
    outbuf[pl.ds(16, L)] = v1
    outbuf[pl.ds(32, L)] = v2
    pltpu.sync_copy(outbuf, out.at[w])


@functools.lru_cache(maxsize=None)
def _sc_call():
    return pl.kernel(
        _body,
        out_type=jax.ShapeDtypeStruct((NW, 48), jnp.float32),
        mesh=plsc.VectorSubcoreMesh(core_axis_name="c", subcore_axis_name="s",
                                    num_cores=NC, num_subcores=NS),
        compiler_params=pltpu.CompilerParams(needs_layout_passes=False, use_tc_tiling_on_sc=False),
        scratch_types=[
            pltpu.VMEM((G,), jnp.float32),
            pltpu.VMEM((CH,), jnp.float32),
            pltpu.VMEM((CH,), jnp.float32),
            pltpu.VMEM((L,), jnp.int32),
            pltpu.VMEM((L, DPAD), jnp.float32),
            pltpu.VMEM((48,), jnp.float32),
            pltpu.SemaphoreType.DMA,
            pltpu.SemaphoreType.DMA,
            pltpu.SemaphoreType.DMA,
            pltpu.SemaphoreType.DMA,
        ],
    )


def kernel(density, cube_shape, grid_xyz, sphere_mask):
    del cube_shape
    B, N, C, _ = density.shape
    dens = density.reshape(CELLS * G)
    maskf = sphere_mask.astype(jnp.float32)
    grid16 = jnp.pad(grid_xyz, ((0, 0), (0, DPAD - 3)))
    out = _sc_call()(dens, maskf, grid16)
    peaks_xyz = out[:, :24].reshape(B, N, C, 2, 3)
    peaks_score = out[:, 24:32].reshape(B, N, C, 2)
    peaks_mask = out[:, 32:40].reshape(B, N, C, 2) > 0.5
    return peaks_xyz, peaks_score, peaks_mask


# speedup vs baseline: 1.0034x; 1.0034x over previous
import functools

import jax
import jax.numpy as jnp
from jax import lax
from jax.experimental import pallas as pl
from jax.experimental.pallas import tpu as pltpu
from jax.experimental.pallas import tpu_sc as plsc

NXYZ = 48
G = NXYZ * NXYZ * NXYZ
CELLS = 128
NC, NS, L = 2, 16, 16
NW = NC * NS
CPW = CELLS // NW
RAD = 4
NEG = -1e9
THRESH = -1e8
NSTEP = G // L
NCH = 16
CH_STEP = NSTEP // NCH
CH = CH_STEP * L
DPAD = 16
BIG = 1 << 30
UNROLL = 4


def _merge(cms, cis):
    cm, ci = cms[0], cis[0]
    for g in range(1, len(cms)):
        m = jnp.maximum(cm, cms[g])
        both = (cm == m) & (cms[g] == m)
        pick_b = (cms[g] == m) & ~(cm == m)
        ni = jnp.where(both, jnp.minimum(ci, cis[g]),
                       jnp.where(pick_b, cis[g], ci))
        cm, ci = m, ni
    return cm, ci


def _body(dens, maskf, grid, out, vbuf, mb0, mb1, idxv, xyzv, outbuf,
          semd, semm0, semm1, semg):
    cid = lax.axis_index("c")
    sid = lax.axis_index("s")
    w = sid * NC + cid
    _worker(w, dens, maskf, grid, out, vbuf, mb0, mb1, idxv, xyzv, outbuf,
            semd, semm0, semm1, semg)


def _worker(w, dens, maskf, grid, out, vbuf, mb0, mb1, idxv, xyzv, outbuf,
            semd, semm0, semm1, semg):
    lane = lax.iota(jnp.int32, L)
    neg = jnp.float32(NEG)
    thresh = jnp.float32(THRESH)
    big = jnp.int32(BIG)
    gidxs, scores, alives = [], [], []

    for t in range(CPW):
        cell = w * CPW + t
        cpd = pltpu.async_copy(dens.at[pl.ds(cell * G, G)], vbuf, semd)
        mcopies = [None, None]
        mcopies[0] = pltpu.async_copy(maskf.at[pl.ds(0, CH)], mb0, semm0)
        cpd.wait()

        curmax = tuple(jnp.full((L,), neg, jnp.float32) for _ in range(UNROLL))
        curidx = tuple(jnp.zeros((L,), jnp.int32) for _ in range(UNROLL))
        for ch in range(NCH):
            mcopies[ch % 2].wait()
            if ch + 1 < NCH:
                mb_next = mb1 if (ch + 1) % 2 else mb0
                sem_next = semm1 if (ch + 1) % 2 else semm0
                mcopies[(ch + 1) % 2] = pltpu.async_copy(
                    maskf.at[pl.ds((ch + 1) * CH, CH)], mb_next, sem_next)
            mb = mb1 if ch % 2 else mb0

            def step(u, carry, _ch=ch, _mb=mb):
                cms, cis = carry
                ncms, ncis = [], []
                for g in range(UNROLL):
                    off = _ch * CH + (u * UNROLL + g) * L
                    moff = (u * UNROLL + g) * L
                    v = vbuf[pl.ds(off, L)]
                    m = _mb[pl.ds(moff, L)]
                    v = jnp.where(m > jnp.float32(0.5), v, neg)
                    vbuf[pl.ds(off, L)] = v
                    gt = v > cms[g]
                    ncms.append(jnp.where(gt, v, cms[g]))
                    ncis.append(jnp.where(gt, lane + off, cis[g]))
                return tuple(ncms), tuple(ncis)

            curmax, curidx = lax.fori_loop(
                0, CH_STEP // UNROLL, step, (curmax, curidx))

        cm1, ci1 = _merge(curmax, curidx)
        gmax = jnp.max(cm1)
        gidx = jnp.min(jnp.where(cm1 == gmax, ci1, big))
        alive1 = gmax >= thresh

        i0 = gidx // (NXYZ * NXYZ)
        rem = gidx - i0 * (NXYZ * NXYZ)
        j0 = rem // NXYZ
        k0 = rem - j0 * NXYZ
        ks = jnp.clip(k0 - RAD, 0, NXYZ - L)
        kwin = ks + lane
        kmask = jnp.abs(kwin - k0) <= RAD

        def supp(s, carry):
            di = s // (2 * RAD + 1) - RAD
            dj = s % (2 * RAD + 1) - RAD
            ii = i0 + di
            jj = j0 + dj
            valid = (ii >= 0) & (ii < NXYZ) & (jj >= 0) & (jj < NXYZ)
            iic = jnp.clip(ii, 0, NXYZ - 1)
            jjc = jnp.clip(jj, 0, NXYZ - 1)
            base = (iic * NXYZ + jjc) * NXYZ + ks
            vv = vbuf[pl.ds(base, L)]
            vbuf[pl.ds(base, L)] = jnp.where(kmask & valid, neg, vv)
            return carry

        lax.fori_loop(0, (2 * RAD + 1) * (2 * RAD + 1), supp, 0)

        def step2(u, carry):
            cms, cis = carry
            ncms, ncis = [], []
            for g in range(UNROLL):
                off = (u * UNROLL + g) * L
                v = vbuf[pl.ds(off, L)]
                gt = v > cms[g]
                ncms.append(jnp.where(gt, v, cms[g]))
                ncis.append(jnp.where(gt, lane + off, cis[g]))
            return tuple(ncms), tuple(ncis)

        curmax2 = tuple(jnp.full((L,), neg, jnp.float32) for _ in range(UNROLL))
        curidx2 = tuple(jnp.zeros((L,), jnp.int32) for _ in range(UNROLL))
        curmax2, curidx2 = lax.fori_loop(0, NSTEP // UNROLL, step2,
                                         (curmax2, curidx2))
        cm2, ci2 = _merge(curmax2, curidx2)
        gmax2 = jnp.max(cm2)
        gidx2 = jnp.min(jnp.where(cm2 == gmax2, ci2, big))
        alive2 = alive1 & (gmax2 >= thresh)

        gidxs.append(jnp.where(alive1, gidx, 0))
        gidxs.append(jnp.where(alive2, gidx2, 0))
        scores.append(jnp.where(alive1, gmax, neg))
        scores.append(jnp.where(alive2, gmax2, neg))
        alives.append(alive1)
        alives.append(alive2)

    idxacc = jnp.zeros((L,), jnp.int32)
    for p in range(2 * CPW):
        idxacc = jnp.where(lane == p, gidxs[p], idxacc)
    idxv[...] = idxacc
    pltpu.async_copy(grid.at[idxv], xyzv, semg).wait()

    afl = [jnp.where(a, jnp.float32(1.0), jnp.float32(0.0)) for a in alives]
    v0 = jnp.zeros((L,), jnp.float32)
    v1 = jnp.zeros((L,), jnp.float32)
    v2 = jnp.zeros((L,), jnp.float32)
    for p in range(2 * CPW):
        row = xyzv[p, :]
        for q in range(3):
            s = 3 * p + q
            val = row[q] * afl[p]
            if s < 16:
                v0 = jnp.where(lane == s, val, v0)
            else:
                v1 = jnp.where(lane == (s - 16), val, v1)
    for p in range(2 * CPW):
        v1 = jnp.where(lane == (8 + p), scores[p], v1)
        v2 = jnp.where(lane == p, afl[p], v2)
    outbuf[pl.ds(0, L)] = v0
    outbuf[pl.ds(16, L)] = v1
    outbuf[pl.ds(32, L)] = v2
    pltpu.sync_copy(outbuf, out.at[w])


@functools.lru_cache(maxsize=None)
def _sc_call():
    return pl.kernel(
        _body,
        out_type=jax.ShapeDtypeStruct((NW, 48), jnp.float32),
        mesh=plsc.VectorSubcoreMesh(core_axis_name="c", subcore_axis_name="s",
                                    num_cores=NC, num_subcores=NS),
        compiler_params=pltpu.CompilerParams(needs_layout_passes=False, use_tc_tiling_on_sc=False),
        scratch_types=[
            pltpu.VMEM((G,), jnp.float32),
            pltpu.VMEM((CH,), jnp.float32),
            pltpu.VMEM((CH,), jnp.float32),
            pltpu.VMEM((L,), jnp.int32),
            pltpu.VMEM((L, DPAD), jnp.float32),
            pltpu.VMEM((48,), jnp.float32),
            pltpu.SemaphoreType.DMA,
            pltpu.SemaphoreType.DMA,
            pltpu.SemaphoreType.DMA,
            pltpu.SemaphoreType.DMA,
        ],
    )


def kernel(density, cube_shape, grid_xyz, sphere_mask):
    del cube_shape
    B, N, C, _ = density.shape
    dens = density.reshape(CELLS * G)
    maskf = sphere_mask.astype(jnp.float32)
    grid16 = jnp.pad(grid_xyz, ((0, 0), (0, DPAD - 3)))
    out = _sc_call()(dens, maskf, grid16)
    peaks_xyz = out[:, :24].reshape(B, N, C, 2, 3)
    peaks_score = out[:, 24:32].reshape(B, N, C, 2)
    peaks_mask = out[:, 32:40].reshape(B, N, C, 2) > 0.5
    return peaks_xyz, peaks_score, peaks_mask


# speedup vs baseline: 1.1534x; 1.1495x over previous
import functools

import jax
import jax.numpy as jnp
from jax import lax
from jax.experimental import pallas as pl
from jax.experimental.pallas import tpu as pltpu
from jax.experimental.pallas import tpu_sc as plsc

NXYZ = 48
G = NXYZ * NXYZ * NXYZ
CELLS = 128
NC, NS, L = 2, 16, 16
NW = NC * NS
CPW = CELLS // NW
RAD = 4
NEG = -1e9
THRESH = -1e8
NSTEP = G // L
NCH = 16
CH_STEP = NSTEP // NCH
CH = CH_STEP * L
DPAD = 16
BIG = 1 << 30
UNROLL = 4


def _merge(cms, cis):
    cm, ci = cms[0], cis[0]
    for g in range(1, len(cms)):
        m = jnp.maximum(cm, cms[g])
        both = (cm == m) & (cms[g] == m)
        pick_b = (cms[g] == m) & ~(cm == m)
        ni = jnp.where(both, jnp.minimum(ci, cis[g]),
                       jnp.where(pick_b, cis[g], ci))
        cm, ci = m, ni
    return cm, ci


def _body(dens, maskf, out, vbuf, mb0, mb1, outbuf,
          semd, semm0, semm1):
    cid = lax.axis_index("c")
    sid = lax.axis_index("s")
    w = sid * NC + cid
    _worker(w, dens, maskf, out, vbuf, mb0, mb1, outbuf,
            semd, semm0, semm1)


def _worker(w, dens, maskf, out, vbuf, mb0, mb1, outbuf,
            semd, semm0, semm1):
    lane = lax.iota(jnp.int32, L)
    neg = jnp.float32(NEG)
    thresh = jnp.float32(THRESH)
    big = jnp.int32(BIG)
    gidxs, scores, alives = [], [], []

    for t in range(CPW):
        cell = w * CPW + t
        cpd = pltpu.async_copy(dens.at[cell], vbuf, semd)
        mcopies = [None, None]
        mcopies[0] = pltpu.async_copy(maskf.at[pl.ds(0, CH)], mb0, semm0)
        cpd.wait()

        curmax = tuple(jnp.full((L,), neg, jnp.float32) for _ in range(UNROLL))
        curidx = tuple(jnp.zeros((L,), jnp.int32) for _ in range(UNROLL))
        for ch in range(NCH):
            mcopies[ch % 2].wait()
            if ch + 1 < NCH:
                mb_next = mb1 if (ch + 1) % 2 else mb0
                sem_next = semm1 if (ch + 1) % 2 else semm0
                mcopies[(ch + 1) % 2] = pltpu.async_copy(
                    maskf.at[pl.ds((ch + 1) * CH, CH)], mb_next, sem_next)
            mb = mb1 if ch % 2 else mb0

            def step(u, carry, _ch=ch, _mb=mb):
                cms, cis = carry
                ncms, ncis = [], []
                for g in range(UNROLL):
                    off = _ch * CH + (u * UNROLL + g) * L
                    moff = (u * UNROLL + g) * L
                    v = vbuf[pl.ds(off, L)]
                    m = _mb[pl.ds(moff, L)]
                    v = jnp.where(m > jnp.float32(0.5), v, neg)
                    vbuf[pl.ds(off, L)] = v
                    gt = v > cms[g]
                    ncms.append(jnp.where(gt, v, cms[g]))
                    ncis.append(jnp.where(gt, lane + off, cis[g]))
                return tuple(ncms), tuple(ncis)

            curmax, curidx = lax.fori_loop(
                0, CH_STEP // UNROLL, step, (curmax, curidx))

        cm1, ci1 = _merge(curmax, curidx)
        gmax = jnp.max(cm1)
        gidx = jnp.min(jnp.where(cm1 == gmax, ci1, big))
        alive1 = gmax >= thresh

        i0 = gidx // (NXYZ * NXYZ)
        rem = gidx - i0 * (NXYZ * NXYZ)
        j0 = rem // NXYZ
        k0 = rem - j0 * NXYZ
        ks = jnp.clip(k0 - RAD, 0, NXYZ - L)
        kwin = ks + lane
        kmask = jnp.abs(kwin - k0) <= RAD

        def supp(s, carry):
            di = s // (2 * RAD + 1) - RAD
            dj = s % (2 * RAD + 1) - RAD
            ii = i0 + di
            jj = j0 + dj
            valid = (ii >= 0) & (ii < NXYZ) & (jj >= 0) & (jj < NXYZ)
            iic = jnp.clip(ii, 0, NXYZ - 1)
            jjc = jnp.clip(jj, 0, NXYZ - 1)
            base = (iic * NXYZ + jjc) * NXYZ + ks
            vv = vbuf[pl.ds(base, L)]
            vbuf[pl.ds(base, L)] = jnp.where(kmask & valid, neg, vv)
            return carry

        lax.fori_loop(0, (2 * RAD + 1) * (2 * RAD + 1), supp, 0)

        def step2(u, carry):
            cms, cis = carry
            ncms, ncis = [], []
            for g in range(UNROLL):
                off = (u * UNROLL + g) * L
                v = vbuf[pl.ds(off, L)]
                gt = v > cms[g]
                ncms.append(jnp.where(gt, v, cms[g]))
                ncis.append(jnp.where(gt, lane + off, cis[g]))
            return tuple(ncms), tuple(ncis)

        curmax2 = tuple(jnp.full((L,), neg, jnp.float32) for _ in range(UNROLL))
        curidx2 = tuple(jnp.zeros((L,), jnp.int32) for _ in range(UNROLL))
        curmax2, curidx2 = lax.fori_loop(0, NSTEP // UNROLL, step2,
                                         (curmax2, curidx2))
        cm2, ci2 = _merge(curmax2, curidx2)
        gmax2 = jnp.max(cm2)
        gidx2 = jnp.min(jnp.where(cm2 == gmax2, ci2, big))
        alive2 = alive1 & (gmax2 >= thresh)

        gidxs.append(jnp.where(alive1, gidx, 0))
        gidxs.append(jnp.where(alive2, gidx2, 0))
        scores.append(jnp.where(alive1, gmax, neg))
        scores.append(jnp.where(alive2, gmax2, neg))
        alives.append(alive1)
        alives.append(alive2)

    afl = [jnp.where(a, jnp.float32(1.0), jnp.float32(0.0)) for a in alives]
    v0 = jnp.zeros((L,), jnp.float32)
    v1 = jnp.zeros((L,), jnp.float32)
    for p in range(2 * CPW):
        v0 = jnp.where(lane == p, scores[p], v0)
        v0 = jnp.where(lane == (8 + p), afl[p], v0)
        v1 = jnp.where(lane == p, gidxs[p].astype(jnp.float32), v1)
    outbuf[pl.ds(0, L)] = v0
    outbuf[pl.ds(16, L)] = v1
    outbuf[pl.ds(32, L)] = jnp.zeros((L,), jnp.float32)
    pltpu.sync_copy(outbuf, out.at[pl.ds(w * 48, 48)])


@functools.lru_cache(maxsize=None)
def _sc_call():
    return pl.kernel(
        _body,
        out_type=jax.ShapeDtypeStruct((NW * 48,), jnp.float32),
        mesh=plsc.VectorSubcoreMesh(core_axis_name="c", subcore_axis_name="s",
                                    num_cores=NC, num_subcores=NS),
        compiler_params=pltpu.CompilerParams(needs_layout_passes=False),
        scratch_types=[
            pltpu.VMEM((G,), jnp.float32),
            pltpu.VMEM((CH,), jnp.float32),
            pltpu.VMEM((CH,), jnp.float32),
            pltpu.VMEM((48,), jnp.float32),
            pltpu.SemaphoreType.DMA,
            pltpu.SemaphoreType.DMA,
            pltpu.SemaphoreType.DMA,
        ],
    )


def kernel(density, cube_shape, grid_xyz, sphere_mask):
    del cube_shape
    B, N, C, _ = density.shape
    dens = density.reshape(CELLS, G)
    maskf = sphere_mask.astype(jnp.float32)
    out = _sc_call()(dens, maskf).reshape(NW, 48)
    peaks_score = out[:, 0:8].reshape(B, N, C, 2)
    alive = out[:, 8:16].reshape(B, N, C, 2)
    idx = out[:, 16:24].astype(jnp.int32).reshape(B, N, C, 2)
    peaks_mask = alive > 0.5
    peaks_xyz = grid_xyz[idx] * alive[..., None]
    return peaks_xyz, peaks_score, peaks_mask


# speedup vs baseline: 1.3838x; 1.1998x over previous
import functools

import jax
import jax.numpy as jnp
from jax import lax
from jax.experimental import pallas as pl
from jax.experimental.pallas import tpu as pltpu
from jax.experimental.pallas import tpu_sc as plsc

NXYZ = 48
NROW = NXYZ * NXYZ
G = NROW * NXYZ
CELLS = 128
NC, NS, L = 2, 16, 16
NW = NC * NS
CPW = CELLS // NW
RAD = 4
NEG = -1e9
THRESH = -1e8
BIG = 1 << 30
GCH = NROW
NQ = G // GCH
RB = 16
NB = GCH // NXYZ // RB
RUN = 2 * RAD + 1
RUNW = RUN * NXYZ


def _row_argk(rowbuf, rowmk, target, lane, big):
    neg = jnp.float32(NEG)
    kbest = big
    for s in range(NXYZ // L):
        v = rowbuf[pl.ds(16 * s, L)]
        m = rowmk[pl.ds(16 * s, L)]
        val = jnp.where(m > jnp.float32(0.5), v, neg)
        kv = jnp.where(val == target, lane + 16 * s, big)
        kbest = jnp.minimum(kbest, jnp.min(kv))
    return kbest


def _body(dens, maskf, out, dstage, mstage, tbuf, rowmax, runbuf, runmk,
          rowbuf, rowmk, outbuf, semd0, semd1, semm0, semm1, semr):
    cid = lax.axis_index("c")
    sid = lax.axis_index("s")
    w = sid * NC + cid
    _worker(w, dens, maskf, out, dstage, mstage, tbuf, rowmax, runbuf,
            runmk, rowbuf, rowmk, outbuf, semd0, semd1, semm0, semm1, semr)


def _worker(w, dens, maskf, out, dstage, mstage, tbuf, rowmax, runbuf,
            runmk, rowbuf, rowmk, outbuf, semd0, semd1, semm0, semm1, semr):
    lane = lax.iota(jnp.int32, L)
    neg = jnp.float32(NEG)
    thresh = jnp.float32(THRESH)
    big = jnp.int32(BIG)
    half = jnp.float32(0.5)
    r0 = w * CPW

    semd = (semd0, semd1)
    semm = (semm0, semm1)

    def issue(q, par):
        for c in range(CPW):
            pltpu.async_copy(dens.at[pl.ds((r0 + c) * G + q * GCH, GCH)],
                             dstage.at[par, c], semd[par])
        pltpu.async_copy(maskf.at[pl.ds(q * GCH, GCH)],
                         mstage.at[par], semm[par])

    def drain(par):
        for c in range(CPW):
            pltpu.make_async_copy(dens.at[pl.ds(0, GCH)],
                                  dstage.at[par, c], semd[par]).wait()
        pltpu.make_async_copy(maskf.at[pl.ds(0, GCH)],
                              mstage.at[par], semm[par]).wait()

    def process(q, par):
        def batch(b, carry):
            rbase = q * (GCH // NXYZ) + b * RB
            for r in range(RB):
                moff = (b * RB + r) * NXYZ
                ms = [mstage[par, pl.ds(moff + 16 * s, L)]
                      for s in range(NXYZ // L)]
                for c in range(CPW):
                    m01 = None
                    for s in range(NXYZ // L):
                        v = dstage[par, c, pl.ds(moff + 16 * s, L)]
                        val = jnp.where(ms[s] > half, v, neg)
                        m01 = val if m01 is None else jnp.maximum(m01, val)
                    tbuf[c, r] = m01
            for c in range(CPW):
                rmv = None
                for l in range(L):
                    col = plsc.load_gather(
                        tbuf, [jnp.full((L,), c, jnp.int32), lane,
                               jnp.full((L,), l, jnp.int32)])
                    rmv = col if rmv is None else jnp.maximum(rmv, col)
                rowmax[c, pl.ds(rbase, RB)] = rmv
            return carry

        lax.fori_loop(0, NB, batch, 0)

    issue(0, 0)
    issue(1, 1)

    def chunk2(qq, carry):
        q0 = 2 * qq
        drain(0)
        process(q0, 0)

        @pl.when(q0 + 2 < NQ)
        def _issue0():
            issue(q0 + 2, 0)

        drain(1)
        process(q0 + 1, 1)

        @pl.when(q0 + 3 < NQ)
        def _issue1():
            issue(q0 + 3, 1)

        return carry

    lax.fori_loop(0, NQ // 2, chunk2, 0)

    gidxs, scores, alives = [], [], []
    for c in range(CPW):
        cell = r0 + c

        def scan1(u, carry):
            rm, rr = carry
            rv = rowmax[c, pl.ds(16 * u, L)]
            gt = rv > rm
            rm = jnp.where(gt, rv, rm)
            rr = jnp.where(gt, lane + 16 * u, rr)
            return rm, rr

        rm, rr = lax.fori_loop(
            0, NROW // L, scan1,
            (jnp.full((L,), neg, jnp.float32), jnp.zeros((L,), jnp.int32)))
        gmax1 = jnp.max(rm)
        r1 = jnp.min(jnp.where(rm == gmax1, rr, big))
        alive1 = gmax1 >= thresh

        cp1 = pltpu.async_copy(
            dens.at[pl.ds(cell * G + r1 * NXYZ, NXYZ)], rowbuf, semr)
        cp2 = pltpu.async_copy(maskf.at[pl.ds(r1 * NXYZ, NXYZ)], rowmk, semr)
        cp1.wait()
        cp2.wait()
        k1 = _row_argk(rowbuf, rowmk, gmax1, lane, big)
        gidx1 = r1 * NXYZ + k1
        i0 = r1 // NXYZ
        j0 = r1 - i0 * NXYZ
        k0 = k1

        jstart = jnp.clip(j0 - RAD, 0, NXYZ - RUN)
        rcopies = []
        for d in range(RUN):
            iic = jnp.clip(i0 - RAD + d, 0, NXYZ - 1)
            base = (iic * NXYZ + jstart) * NXYZ
            rcopies.append(pltpu.async_copy(
                dens.at[pl.ds(cell * G + base, RUNW)], runbuf.at[d], semr))
            rcopies.append(pltpu.async_copy(
                maskf.at[pl.ds(base, RUNW)], runmk.at[d], semr))

        def scan2(u, carry):
            rm2, rr2 = carry
            rv = rowmax[c, pl.ds(16 * u, L)]
            rows = lane + 16 * u
            iv = rows // NXYZ
            jv = rows - iv * NXYZ
            inbox = ((jnp.abs(iv - i0) <= RAD) & (jnp.abs(jv - j0) <= RAD))
            rv = jnp.where(inbox, neg, rv)
            gt = rv > rm2
            rm2 = jnp.where(gt, rv, rm2)
            rr2 = jnp.where(gt, rows, rr2)
            return rm2, rr2

        rm2, rr2 = lax.fori_loop(
            0, NROW // L, scan2,
            (jnp.full((L,), neg, jnp.float32), jnp.zeros((L,), jnp.int32)))
        smax = jnp.max(rm2)
        srow = jnp.min(jnp.where(rm2 == smax, rr2, big))

        for cp in rcopies:
            cp.wait()

        rcm = jnp.full((L,), neg, jnp.float32)
        rci = jnp.zeros((L,), jnp.int32)
        for d in range(RUN):
            ii = i0 - RAD + d
            valid_i = (ii >= 0) & (ii < NXYZ)
            iic = jnp.clip(ii, 0, NXYZ - 1)
            fbase = iic * NROW + jstart * NXYZ

            def run_step(u, carry, _d=d, _fbase=fbase, _valid=valid_i):
                cm, ci = carry
                gl = lane + 16 * u
                jloc = gl // NXYZ
                kv = gl - jloc * NXYZ
                jv = jstart + jloc
                supp = ((jnp.abs(kv - k0) <= RAD)
                        & (jnp.abs(jv - j0) <= RAD))
                v = runbuf[_d, pl.ds(16 * u, L)]
                m = runmk[_d, pl.ds(16 * u, L)]
                val = jnp.where(m > half, v, neg)
                val = jnp.where(supp | ~_valid, neg, val)
                gt = val > cm
                cm = jnp.where(gt, val, cm)
                ci = jnp.where(gt, _fbase + gl, ci)
                return cm, ci

            rcm, rci = lax.fori_loop(0, RUNW // L, run_step, (rcm, rci))
        rmax = jnp.max(rcm)
        ridx = jnp.min(jnp.where(rcm == rmax, rci, big))

        gmax2 = jnp.maximum(smax, rmax)
        alive2 = alive1 & (gmax2 >= thresh)

        cp1 = pltpu.async_copy(
            dens.at[pl.ds(cell * G + srow * NXYZ, NXYZ)], rowbuf, semr)
        cp2 = pltpu.async_copy(maskf.at[pl.ds(srow * NXYZ, NXYZ)],
                               rowmk, semr)
        cp1.wait()
        cp2.wait()
        kS = _row_argk(rowbuf, rowmk, gmax2, lane, big)
        idxS = srow * NXYZ + kS

        gidx2 = jnp.where(smax > rmax, idxS,
                          jnp.where(rmax > smax, ridx,
                                    jnp.minimum(idxS, ridx)))

        gidxs.append(jnp.where(alive1, gidx1, 0))
        gidxs.append(jnp.where(alive2, gidx2, 0))
        scores.append(jnp.where(alive1, gmax1, neg))
        scores.append(jnp.where(alive2, gmax2, neg))
        alives.append(alive1)
        alives.append(alive2)

    afl = [jnp.where(a, jnp.float32(1.0), jnp.float32(0.0)) for a in alives]
    v0 = jnp.zeros((L,), jnp.float32)
    v1 = jnp.zeros((L,), jnp.float32)
    for p in range(2 * CPW):
        v0 = jnp.where(lane == p, scores[p], v0)
        v0 = jnp.where(lane == (8 + p), afl[p], v0)
        v1 = jnp.where(lane == p, gidxs[p].astype(jnp.float32), v1)
    outbuf[pl.ds(0, L)] = v0
    outbuf[pl.ds(16, L)] = v1
    outbuf[pl.ds(32, L)] = jnp.zeros((L,), jnp.float32)
    pltpu.sync_copy(outbuf, out.at[pl.ds(w * 48, 48)])


@functools.lru_cache(maxsize=None)
def _sc_call():
    return pl.kernel(
        _body,
        out_type=jax.ShapeDtypeStruct((NW * 48,), jnp.float32),
        mesh=plsc.VectorSubcoreMesh(core_axis_name="c", subcore_axis_name="s",
                                    num_cores=NC, num_subcores=NS),
        compiler_params=pltpu.CompilerParams(needs_layout_passes=False, use_tc_tiling_on_sc=False),
        scratch_types=[
            pltpu.VMEM((2, CPW, GCH), jnp.float32),
            pltpu.VMEM((2, GCH), jnp.float32),
            pltpu.VMEM((CPW, RB, L), jnp.float32),
            pltpu.VMEM((CPW, NROW), jnp.float32),
            pltpu.VMEM((RUN, RUNW), jnp.float32),
            pltpu.VMEM((RUN, RUNW), jnp.float32),
            pltpu.VMEM((NXYZ,), jnp.float32),
            pltpu.VMEM((NXYZ,), jnp.float32),
            pltpu.VMEM((48,), jnp.float32),
            pltpu.SemaphoreType.DMA,
            pltpu.SemaphoreType.DMA,
            pltpu.SemaphoreType.DMA,
            pltpu.SemaphoreType.DMA,
            pltpu.SemaphoreType.DMA,
        ],
    )


def kernel(density, cube_shape, grid_xyz, sphere_mask):
    del cube_shape
    B, N, C, _ = density.shape
    dens = density.reshape(CELLS * G)
    maskf = sphere_mask.astype(jnp.float32)
    out = _sc_call()(dens, maskf).reshape(NW, 48)
    peaks_score = out[:, 0:8].reshape(B, N, C, 2)
    alive = out[:, 8:16].reshape(B, N, C, 2)
    idx = out[:, 16:24].astype(jnp.int32).reshape(B, N, C, 2)
    peaks_mask = alive > 0.5
    peaks_xyz = grid_xyz[idx] * alive[..., None]
    return peaks_xyz, peaks_score, peaks_mask


# speedup vs baseline: 1.4029x; 1.0138x over previous
import functools

import jax
import jax.numpy as jnp
from jax import lax
from jax.experimental import pallas as pl
from jax.experimental.pallas import tpu as pltpu
from jax.experimental.pallas import tpu_sc as plsc

NXYZ = 48
NROW = NXYZ * NXYZ
G = NROW * NXYZ
CELLS = 128
NC, NS, L = 2, 16, 16
NW = NC * NS
CPW = CELLS // NW
RAD = 4
NEG = -1e9
THRESH = -1e8
BIG = 1 << 30
GCH = NROW
NQ = G // GCH
RB = 16
NB = GCH // NXYZ // RB
RUN = 2 * RAD + 1
RUNW = RUN * NXYZ


def _row_argk(rowbuf, rowmk, target, lane, big):
    neg = jnp.float32(NEG)
    kbest = big
    for s in range(NXYZ // L):
        v = rowbuf[pl.ds(16 * s, L)]
        m = rowmk[pl.ds(16 * s, L)]
        val = jnp.where(m > jnp.float32(0.5), v, neg)
        kv = jnp.where(val == target, lane + 16 * s, big)
        kbest = jnp.minimum(kbest, jnp.min(kv))
    return kbest


def _merge(pairs):
    cm, ci = pairs[0]
    for cmb, cib in pairs[1:]:
        m = jnp.maximum(cm, cmb)
        eqa = cm == m
        eqb = cmb == m
        ci = jnp.where(eqa & eqb, jnp.minimum(ci, cib),
                       jnp.where(eqb & ~eqa, cib, ci))
        cm = m
    return cm, ci


def _body(dens, maskf, out, dstage, mstage, tbuf, rowmax, runbuf, runmk,
          rowbuf, rowmk, outbuf, semd0, semd1, semm0, semm1, semr):
    cid = lax.axis_index("c")
    sid = lax.axis_index("s")
    w = sid * NC + cid
    _worker(w, dens, maskf, out, dstage, mstage, tbuf, rowmax, runbuf,
            runmk, rowbuf, rowmk, outbuf, semd0, semd1, semm0, semm1, semr)


def _worker(w, dens, maskf, out, dstage, mstage, tbuf, rowmax, runbuf,
            runmk, rowbuf, rowmk, outbuf, semd0, semd1, semm0, semm1, semr):
    lane = lax.iota(jnp.int32, L)
    neg = jnp.float32(NEG)
    thresh = jnp.float32(THRESH)
    big = jnp.int32(BIG)
    half = jnp.float32(0.5)
    r0 = w * CPW

    semd = (semd0, semd1)
    semm = (semm0, semm1)

    def issue(q, par):
        for c in range(CPW):
            pltpu.async_copy(dens.at[pl.ds((r0 + c) * G + q * GCH, GCH)],
                             dstage.at[par, c], semd[par])
        pltpu.async_copy(maskf.at[pl.ds(q * GCH, GCH)],
                         mstage.at[par], semm[par])

    def drain(par):
        for c in range(CPW):
            pltpu.make_async_copy(dens.at[pl.ds(0, GCH)],
                                  dstage.at[par, c], semd[par]).wait()
        pltpu.make_async_copy(maskf.at[pl.ds(0, GCH)],
                              mstage.at[par], semm[par]).wait()

    def process(q, par):
        def batch(b, carry):
            rbase = q * (GCH // NXYZ) + b * RB
            for r in range(RB):
                moff = (b * RB + r) * NXYZ
                ms = [mstage[par, pl.ds(moff + 16 * s, L)]
                      for s in range(NXYZ // L)]
                for c in range(CPW):
                    m01 = None
                    for s in range(NXYZ // L):
                        v = dstage[par, c, pl.ds(moff + 16 * s, L)]
                        val = jnp.where(ms[s] > half, v, neg)
                        m01 = val if m01 is None else jnp.maximum(m01, val)
                    tbuf[c, r] = m01
            for c in range(CPW):
                rmv = None
                for l in range(L):
                    col = plsc.load_gather(
                        tbuf, [jnp.full((L,), c, jnp.int32), lane,
                               jnp.full((L,), l, jnp.int32)])
                    rmv = col if rmv is None else jnp.maximum(rmv, col)
                rowmax[c, pl.ds(rbase, RB)] = rmv
            return carry

        lax.fori_loop(0, NB, batch, 0)

    issue(0, 0)
    issue(1, 1)

    def chunk2(qq, carry):
        q0 = 2 * qq
        drain(0)
        process(q0, 0)

        @pl.when(q0 + 2 < NQ)
        def _issue0():
            issue(q0 + 2, 0)

        drain(1)
        process(q0 + 1, 1)

        @pl.when(q0 + 3 < NQ)
        def _issue1():
            issue(q0 + 3, 1)

        return carry

    lax.fori_loop(0, NQ // 2, chunk2, 0)

    gidxs, scores, alives = [], [], []
    for c in range(CPW):
        cell = r0 + c

        def scan1(u, carry):
            rms, rrs = carry
            nm, nr = [], []
            for g in range(4):
                rv = rowmax[c, pl.ds(16 * (4 * u + g), L)]
                gt = rv > rms[g]
                nm.append(jnp.where(gt, rv, rms[g]))
                nr.append(jnp.where(gt, lane + 16 * (4 * u + g), rrs[g]))
            return tuple(nm), tuple(nr)

        rms, rrs = lax.fori_loop(
            0, NROW // L // 4, scan1,
            (tuple(jnp.full((L,), neg, jnp.float32) for _ in range(4)),
             tuple(jnp.zeros((L,), jnp.int32) for _ in range(4))))
        rm, rr = _merge(list(zip(rms, rrs)))
        gmax1 = jnp.max(rm)
        r1 = jnp.min(jnp.where(rm == gmax1, rr, big))
        alive1 = gmax1 >= thresh

        cp1 = pltpu.async_copy(
            dens.at[pl.ds(cell * G + r1 * NXYZ, NXYZ)], rowbuf, semr)
        cp2 = pltpu.async_copy(maskf.at[pl.ds(r1 * NXYZ, NXYZ)], rowmk,
                               semr)
        cp1.wait()
        cp2.wait()
        k1 = _row_argk(rowbuf, rowmk, gmax1, lane, big)
        gidx1 = r1 * NXYZ + k1
        i0 = r1 // NXYZ
        j0 = r1 - i0 * NXYZ
        k0 = k1

        jstart = jnp.clip(j0 - RAD, 0, NXYZ - RUN)
        rcopies = []
        for d in range(RUN):
            iic = jnp.clip(i0 - RAD + d, 0, NXYZ - 1)
            base = (iic * NXYZ + jstart) * NXYZ
            rcopies.append(pltpu.async_copy(
                dens.at[pl.ds(cell * G + base, RUNW)], runbuf.at[d], semr))
            rcopies.append(pltpu.async_copy(
                maskf.at[pl.ds(base, RUNW)], runmk.at[d], semr))

        def scan2(u, carry):
            rms2, rrs2 = carry
            nm, nr = [], []
            for g in range(4):
                rv = rowmax[c, pl.ds(16 * (4 * u + g), L)]
                rows = lane + 16 * (4 * u + g)
                iv = rows // NXYZ
                jv = rows - iv * NXYZ
                inbox = ((jnp.abs(iv - i0) <= RAD)
                         & (jnp.abs(jv - j0) <= RAD))
                rv = jnp.where(inbox, neg, rv)
                gt = rv > rms2[g]
                nm.append(jnp.where(gt, rv, rms2[g]))
                nr.append(jnp.where(gt, rows, rrs2[g]))
            return tuple(nm), tuple(nr)

        rms2, rrs2 = lax.fori_loop(
            0, NROW // L // 4, scan2,
            (tuple(jnp.full((L,), neg, jnp.float32) for _ in range(4)),
             tuple(jnp.zeros((L,), jnp.int32) for _ in range(4))))
        rm2, rr2 = _merge(list(zip(rms2, rrs2)))
        smax = jnp.max(rm2)
        srow = jnp.min(jnp.where(rm2 == smax, rr2, big))

        for cp in rcopies:
            cp.wait()

        rcm = jnp.full((L,), neg, jnp.float32)
        rci = jnp.zeros((L,), jnp.int32)
        for d in range(RUN):
            ii = i0 - RAD + d
            valid_i = (ii >= 0) & (ii < NXYZ)
            iic = jnp.clip(ii, 0, NXYZ - 1)
            fbase = iic * NROW + jstart * NXYZ

            def run_step(u, carry, _d=d, _fbase=fbase, _valid=valid_i):
                cm, ci = carry
                for g in range(3):
                    gl = lane + 16 * (3 * u + g)
                    jloc = gl // NXYZ
                    kv = gl - jloc * NXYZ
                    jv = jstart + jloc
                    supp = ((jnp.abs(kv - k0) <= RAD)
                            & (jnp.abs(jv - j0) <= RAD))
                    v = runbuf[_d, pl.ds(16 * (3 * u + g), L)]
                    m = runmk[_d, pl.ds(16 * (3 * u + g), L)]
                    val = jnp.where(m > half, v, neg)
                    val = jnp.where(supp | ~_valid, neg, val)
                    gt = val > cm
                    cm = jnp.where(gt, val, cm)
                    ci = jnp.where(gt, _fbase + gl, ci)
                return cm, ci

            rcm, rci = lax.fori_loop(0, RUNW // L // 3, run_step,
                                     (rcm, rci))
        rmax = jnp.max(rcm)
        ridx = jnp.min(jnp.where(rcm == rmax, rci, big))

        gmax2 = jnp.maximum(smax, rmax)
        alive2 = alive1 & (gmax2 >= thresh)

        cp1 = pltpu.async_copy(
            dens.at[pl.ds(cell * G + srow * NXYZ, NXYZ)], rowbuf, semr)
        cp2 = pltpu.async_copy(maskf.at[pl.ds(srow * NXYZ, NXYZ)], rowmk,
                               semr)
        cp1.wait()
        cp2.wait()
        kS = _row_argk(rowbuf, rowmk, gmax2, lane, big)
        idxS = srow * NXYZ + kS

        gidx2 = jnp.where(smax > rmax, idxS,
                          jnp.where(rmax > smax, ridx,
                                    jnp.minimum(idxS, ridx)))

        gidxs.append(jnp.where(alive1, gidx1, 0))
        gidxs.append(jnp.where(alive2, gidx2, 0))
        scores.append(jnp.where(alive1, gmax1, neg))
        scores.append(jnp.where(alive2, gmax2, neg))
        alives.append(alive1)
        alives.append(alive2)

    afl = [jnp.where(a, jnp.float32(1.0), jnp.float32(0.0)) for a in alives]
    v0 = jnp.zeros((L,), jnp.float32)
    v1 = jnp.zeros((L,), jnp.float32)
    for p in range(2 * CPW):
        v0 = jnp.where(lane == p, scores[p], v0)
        v0 = jnp.where(lane == (8 + p), afl[p], v0)
        v1 = jnp.where(lane == p, gidxs[p].astype(jnp.float32), v1)
    outbuf[pl.ds(0, L)] = v0
    outbuf[pl.ds(16, L)] = v1
    outbuf[pl.ds(32, L)] = jnp.zeros((L,), jnp.float32)
    pltpu.sync_copy(outbuf, out.at[pl.ds(w * 48, 48)])


@functools.lru_cache(maxsize=None)
def _sc_call():
    return pl.kernel(
        _body,
        out_type=jax.ShapeDtypeStruct((NW * 48,), jnp.float32),
        mesh=plsc.VectorSubcoreMesh(core_axis_name="c", subcore_axis_name="s",
                                    num_cores=NC, num_subcores=NS),
        compiler_params=pltpu.CompilerParams(needs_layout_passes=False, use_tc_tiling_on_sc=False),
        scratch_types=[
            pltpu.VMEM((2, CPW, GCH), jnp.float32),
            pltpu.VMEM((2, GCH), jnp.float32),
            pltpu.VMEM((CPW, RB, L), jnp.float32),
            pltpu.VMEM((CPW, NROW), jnp.float32),
            pltpu.VMEM((RUN, RUNW), jnp.float32),
            pltpu.VMEM((RUN, RUNW), jnp.float32),
            pltpu.VMEM((NXYZ,), jnp.float32),
            pltpu.VMEM((NXYZ,), jnp.float32),
            pltpu.VMEM((48,), jnp.float32),
            pltpu.SemaphoreType.DMA,
            pltpu.SemaphoreType.DMA,
            pltpu.SemaphoreType.DMA,
            pltpu.SemaphoreType.DMA,
            pltpu.SemaphoreType.DMA,
        ],
    )


def kernel(density, cube_shape, grid_xyz, sphere_mask):
    del cube_shape
    B, N, C, _ = density.shape
    dens = density.reshape(CELLS * G)
    maskf = sphere_mask.astype(jnp.float32)
    out = _sc_call()(dens, maskf).reshape(NW, 48)
    peaks_score = out[:, 0:8].reshape(B, N, C, 2)
    alive = out[:, 8:16].reshape(B, N, C, 2)
    idx = out[:, 16:24].astype(jnp.int32).reshape(B, N, C, 2)
    peaks_mask = alive > 0.5
    peaks_xyz = grid_xyz[idx] * alive[..., None]
    return peaks_xyz, peaks_score, peaks_mask


# speedup vs baseline: 1.4077x; 1.0034x over previous
import functools

import jax
import jax.numpy as jnp
from jax import lax
from jax.experimental import pallas as pl
from jax.experimental.pallas import tpu as pltpu
from jax.experimental.pallas import tpu_sc as plsc

NXYZ = 48
NROW = NXYZ * NXYZ
G = NROW * NXYZ
CELLS = 128
NC, NS, L = 2, 16, 16
NW = NC * NS
CPW = CELLS // NW
RAD = 4
NEG = -1e9
THRESH = -1e8
BIG = 1 << 30
GCH = 2 * NROW
NQ = G // GCH
RB = 16
NB = GCH // NXYZ // RB
RUN = 2 * RAD + 1
RUNW = RUN * NXYZ


def _row_argk(rowbuf, rowmk, target, lane, big):
    neg = jnp.float32(NEG)
    kbest = big
    for s in range(NXYZ // L):
        v = rowbuf[pl.ds(16 * s, L)]
        m = rowmk[pl.ds(16 * s, L)]
        val = jnp.where(m > jnp.float32(0.5), v, neg)
        kv = jnp.where(val == target, lane + 16 * s, big)
        kbest = jnp.minimum(kbest, jnp.min(kv))
    return kbest


def _merge(pairs):
    cm, ci = pairs[0]
    for cmb, cib in pairs[1:]:
        m = jnp.maximum(cm, cmb)
        eqa = cm == m
        eqb = cmb == m
        ci = jnp.where(eqa & eqb, jnp.minimum(ci, cib),
                       jnp.where(eqb & ~eqa, cib, ci))
        cm = m
    return cm, ci


def _body(dens, maskf, out, dstage, mstage, tbuf, rowmax, runbuf, runmk,
          rowbuf, rowmk, outbuf, semd0, semd1, semm0, semm1, semr):
    cid = lax.axis_index("c")
    sid = lax.axis_index("s")
    w = sid * NC + cid
    _worker(w, dens, maskf, out, dstage, mstage, tbuf, rowmax, runbuf,
            runmk, rowbuf, rowmk, outbuf, semd0, semd1, semm0, semm1, semr)


def _worker(w, dens, maskf, out, dstage, mstage, tbuf, rowmax, runbuf,
            runmk, rowbuf, rowmk, outbuf, semd0, semd1, semm0, semm1, semr):
    lane = lax.iota(jnp.int32, L)
    neg = jnp.float32(NEG)
    thresh = jnp.float32(THRESH)
    big = jnp.int32(BIG)
    half = jnp.float32(0.5)
    r0 = w * CPW

    semd = (semd0, semd1)
    semm = (semm0, semm1)

    def issue(q, par):
        for c in range(CPW):
            pltpu.async_copy(dens.at[pl.ds((r0 + c) * G + q * GCH, GCH)],
                             dstage.at[par, c], semd[par])
        pltpu.async_copy(maskf.at[pl.ds(q * GCH, GCH)],
                         mstage.at[par], semm[par])

    def drain(par):
        for c in range(CPW):
            pltpu.make_async_copy(dens.at[pl.ds(0, GCH)],
                                  dstage.at[par, c], semd[par]).wait()
        pltpu.make_async_copy(maskf.at[pl.ds(0, GCH)],
                              mstage.at[par], semm[par]).wait()

    def process(q, par):
        def batch(b, carry):
            rbase = q * (GCH // NXYZ) + b * RB
            for r in range(RB):
                moff = (b * RB + r) * NXYZ
                ms = [mstage[par, pl.ds(moff + 16 * s, L)]
                      for s in range(NXYZ // L)]
                for c in range(CPW):
                    m01 = None
                    for s in range(NXYZ // L):
                        v = dstage[par, c, pl.ds(moff + 16 * s, L)]
                        val = jnp.where(ms[s] > half, v, neg)
                        m01 = val if m01 is None else jnp.maximum(m01, val)
                    tbuf[c, r] = m01
            for c in range(CPW):
                rmv = None
                for l in range(L):
                    col = plsc.load_gather(
                        tbuf, [jnp.full((L,), c, jnp.int32), lane,
                               jnp.full((L,), l, jnp.int32)])
                    rmv = col if rmv is None else jnp.maximum(rmv, col)
                rowmax[c, pl.ds(rbase, RB)] = rmv
            return carry

        lax.fori_loop(0, NB, batch, 0)

    issue(0, 0)
    issue(1, 1)

    def chunk2(qq, carry):
        q0 = 2 * qq
        drain(0)
        process(q0, 0)

        @pl.when(q0 + 2 < NQ)
        def _issue0():
            issue(q0 + 2, 0)

        drain(1)
        process(q0 + 1, 1)

        @pl.when(q0 + 3 < NQ)
        def _issue1():
            issue(q0 + 3, 1)

        return carry

    lax.fori_loop(0, NQ // 2, chunk2, 0)

    gidxs, scores, alives = [], [], []
    for c in range(CPW):
        cell = r0 + c

        def scan1(u, carry):
            rms, rrs = carry
            nm, nr = [], []
            for g in range(4):
                rv = rowmax[c, pl.ds(16 * (4 * u + g), L)]
                gt = rv > rms[g]
                nm.append(jnp.where(gt, rv, rms[g]))
                nr.append(jnp.where(gt, lane + 16 * (4 * u + g), rrs[g]))
            return tuple(nm), tuple(nr)

        rms, rrs = lax.fori_loop(
            0, NROW // L // 4, scan1,
            (tuple(jnp.full((L,), neg, jnp.float32) for _ in range(4)),
             tuple(jnp.zeros((L,), jnp.int32) for _ in range(4))))
        rm, rr = _merge(list(zip(rms, rrs)))
        gmax1 = jnp.max(rm)
        r1 = jnp.min(jnp.where(rm == gmax1, rr, big))
        alive1 = gmax1 >= thresh

        cp1 = pltpu.async_copy(
            dens.at[pl.ds(cell * G + r1 * NXYZ, NXYZ)], rowbuf, semr)
        cp2 = pltpu.async_copy(maskf.at[pl.ds(r1 * NXYZ, NXYZ)], rowmk,
                               semr)
        cp1.wait()
        cp2.wait()
        k1 = _row_argk(rowbuf, rowmk, gmax1, lane, big)
        gidx1 = r1 * NXYZ + k1
        i0 = r1 // NXYZ
        j0 = r1 - i0 * NXYZ
        k0 = k1

        jstart = jnp.clip(j0 - RAD, 0, NXYZ - RUN)
        rcopies = []
        for d in range(RUN):
            iic = jnp.clip(i0 - RAD + d, 0, NXYZ - 1)
            base = (iic * NXYZ + jstart) * NXYZ
            rcopies.append(pltpu.async_copy(
                dens.at[pl.ds(cell * G + base, RUNW)], runbuf.at[d], semr))
            rcopies.append(pltpu.async_copy(
                maskf.at[pl.ds(base, RUNW)], runmk.at[d], semr))

        def scan2(u, carry):
            rms2, rrs2 = carry
            nm, nr = [], []
            for g in range(4):
                rv = rowmax[c, pl.ds(16 * (4 * u + g), L)]
                rows = lane + 16 * (4 * u + g)
                iv = rows // NXYZ
                jv = rows - iv * NXYZ
                inbox = ((jnp.abs(iv - i0) <= RAD)
                         & (jnp.abs(jv - j0) <= RAD))
                rv = jnp.where(inbox, neg, rv)
                gt = rv > rms2[g]
                nm.append(jnp.where(gt, rv, rms2[g]))
                nr.append(jnp.where(gt, rows, rrs2[g]))
            return tuple(nm), tuple(nr)

        rms2, rrs2 = lax.fori_loop(
            0, NROW // L // 4, scan2,
            (tuple(jnp.full((L,), neg, jnp.float32) for _ in range(4)),
             tuple(jnp.zeros((L,), jnp.int32) for _ in range(4))))
        rm2, rr2 = _merge(list(zip(rms2, rrs2)))
        smax = jnp.max(rm2)
        srow = jnp.min(jnp.where(rm2 == smax, rr2, big))

        for cp in rcopies:
            cp.wait()

        rcm = jnp.full((L,), neg, jnp.float32)
        rci = jnp.zeros((L,), jnp.int32)
        for d in range(RUN):
            ii = i0 - RAD + d
            valid_i = (ii >= 0) & (ii < NXYZ)
            iic = jnp.clip(ii, 0, NXYZ - 1)
            fbase = iic * NROW + jstart * NXYZ

            def run_step(u, carry, _d=d, _fbase=fbase, _valid=valid_i):
                cm, ci = carry
                for g in range(3):
                    gl = lane + 16 * (3 * u + g)
                    jloc = gl // NXYZ
                    kv = gl - jloc * NXYZ
                    jv = jstart + jloc
                    supp = ((jnp.abs(kv - k0) <= RAD)
                            & (jnp.abs(jv - j0) <= RAD))
                    v = runbuf[_d, pl.ds(16 * (3 * u + g), L)]
                    m = runmk[_d, pl.ds(16 * (3 * u + g), L)]
                    val = jnp.where(m > half, v, neg)
                    val = jnp.where(supp | ~_valid, neg, val)
                    gt = val > cm
                    cm = jnp.where(gt, val, cm)
                    ci = jnp.where(gt, _fbase + gl, ci)
                return cm, ci

            rcm, rci = lax.fori_loop(0, RUNW // L // 3, run_step,
                                     (rcm, rci))
        rmax = jnp.max(rcm)
        ridx = jnp.min(jnp.where(rcm == rmax, rci, big))

        gmax2 = jnp.maximum(smax, rmax)
        alive2 = alive1 & (gmax2 >= thresh)

        cp1 = pltpu.async_copy(
            dens.at[pl.ds(cell * G + srow * NXYZ, NXYZ)], rowbuf, semr)
        cp2 = pltpu.async_copy(maskf.at[pl.ds(srow * NXYZ, NXYZ)], rowmk,
                               semr)
        cp1.wait()
        cp2.wait()
        kS = _row_argk(rowbuf, rowmk, gmax2, lane, big)
        idxS = srow * NXYZ + kS

        gidx2 = jnp.where(smax > rmax, idxS,
                          jnp.where(rmax > smax, ridx,
                                    jnp.minimum(idxS, ridx)))

        gidxs.append(jnp.where(alive1, gidx1, 0))
        gidxs.append(jnp.where(alive2, gidx2, 0))
        scores.append(jnp.where(alive1, gmax1, neg))
        scores.append(jnp.where(alive2, gmax2, neg))
        alives.append(alive1)
        alives.append(alive2)

    afl = [jnp.where(a, jnp.float32(1.0), jnp.float32(0.0)) for a in alives]
    v0 = jnp.zeros((L,), jnp.float32)
    v1 = jnp.zeros((L,), jnp.float32)
    for p in range(2 * CPW):
        v0 = jnp.where(lane == p, scores[p], v0)
        v0 = jnp.where(lane == (8 + p), afl[p], v0)
        v1 = jnp.where(lane == p, gidxs[p].astype(jnp.float32), v1)
    outbuf[pl.ds(0, L)] = v0
    outbuf[pl.ds(16, L)] = v1
    outbuf[pl.ds(32, L)] = jnp.zeros((L,), jnp.float32)
    pltpu.sync_copy(outbuf, out.at[pl.ds(w * 48, 48)])


@functools.lru_cache(maxsize=None)
def _sc_call():
    return pl.kernel(
        _body,
        out_type=jax.ShapeDtypeStruct((NW * 48,), jnp.float32),
        mesh=plsc.VectorSubcoreMesh(core_axis_name="c", subcore_axis_name="s",
                                    num_cores=NC, num_subcores=NS),
        compiler_params=pltpu.CompilerParams(needs_layout_passes=False, use_tc_tiling_on_sc=False),
        scratch_types=[
            pltpu.VMEM((2, CPW, GCH), jnp.float32),
            pltpu.VMEM((2, GCH), jnp.float32),
            pltpu.VMEM((CPW, RB, L), jnp.float32),
            pltpu.VMEM((CPW, NROW), jnp.float32),
            pltpu.VMEM((RUN, RUNW), jnp.float32),
            pltpu.VMEM((RUN, RUNW), jnp.float32),
            pltpu.VMEM((NXYZ,), jnp.float32),
            pltpu.VMEM((NXYZ,), jnp.float32),
            pltpu.VMEM((48,), jnp.float32),
            pltpu.SemaphoreType.DMA,
            pltpu.SemaphoreType.DMA,
            pltpu.SemaphoreType.DMA,
            pltpu.SemaphoreType.DMA,
            pltpu.SemaphoreType.DMA,
        ],
    )


def kernel(density, cube_shape, grid_xyz, sphere_mask):
    del cube_shape
    B, N, C, _ = density.shape
    dens = density.reshape(CELLS * G)
    maskf = sphere_mask.astype(jnp.float32)
    out = _sc_call()(dens, maskf).reshape(NW, 48)
    peaks_score = out[:, 0:8].reshape(B, N, C, 2)
    alive = out[:, 8:16].reshape(B, N, C, 2)
    idx = out[:, 16:24].astype(jnp.int32).reshape(B, N, C, 2)
    peaks_mask = alive > 0.5
    peaks_xyz = grid_xyz[idx] * alive[..., None]
    return peaks_xyz, peaks_score, peaks_mask
